# baseline (speedup 1.0000x reference)
import functools

import jax
import jax.numpy as jnp
from jax import lax
from jax.experimental import pallas as pl
from jax.experimental.pallas import tpu as pltpu
from jax.experimental.pallas import tpu_sc as plsc

T = 12
N = 10000
E = 320000
F = 128
NP = 10240
NCORE = 2
NSUB = 16
NW = NCORE * NSUB
C = 96
NCK = 105
EP = NW * C * NCK
EDT = EP // NW
RPT = NP // NSUB
RB = 400
WC = 2000
WR = T * E // WC
HP = lax.Precision.HIGHEST


def _mesh():
    return plsc.VectorSubcoreMesh(core_axis_name="c", subcore_axis_name="s")


def _zero_ref(ref, nrow, ncol):
    def z(i, _):
        for j in range(ncol // 16):
            ref[i, pl.ds(16 * j, 16)] = jnp.zeros((16,), jnp.float32)
        return 0
    lax.fori_loop(0, nrow, z, 0)


def _deg_sc(srcf, wnegf, Tn):
    @functools.partial(
        pl.kernel, mesh=_mesh(),
        out_type=jax.ShapeDtypeStruct((Tn * 2 * NP,), jnp.float32),
        scratch_types=[
            pltpu.VMEM_SHARED((NSUB, NP), jnp.float32),
            pltpu.VMEM((NP + 16,), jnp.float32),
            pltpu.VMEM((EDT,), jnp.int32),
            pltpu.VMEM((EDT,), jnp.float32),
            pltpu.VMEM((RPT,), jnp.float32),
            pltpu.VMEM((RPT,), jnp.float32),
        ],
    )
    def deg(src, wneg, out, stage, hist, idxv, wv, racc, rin):
        cid = lax.axis_index("c")
        sid = lax.axis_index("s")
        wid = sid * NCORE + cid
        m0 = (lax.iota(jnp.int32, 16) == 0).astype(jnp.float32)

        def per_t(t, _):
            def zh(i, _):
                hist[pl.ds(16 * i, 16)] = jnp.zeros((16,), jnp.float32)
                return 0
            lax.fori_loop(0, (NP + 16) // 16, zh, 0)
            base = t * EP + wid * EDT
            pltpu.sync_copy(src.at[pl.ds(base, EDT)], idxv)
            pltpu.sync_copy(wneg.at[pl.ds(base, EDT)], wv)

            def accg(g, _):
                ivec = idxv[pl.ds(16 * g, 16)]
                wvec = wv[pl.ds(16 * g, 16)]
                for l in range(16):
                    ix = ivec[l]
                    hist[pl.ds(ix, 16)] = hist[pl.ds(ix, 16)] + wvec[l] * m0
                return 0
            lax.fori_loop(0, EDT // 16, accg, 0)
            pltpu.sync_copy(hist.at[pl.ds(0, NP)], stage.at[sid])
            plsc.subcore_barrier()
            rb = sid * RPT

            def zr(i, _):
                racc[pl.ds(16 * i, 16)] = jnp.zeros((16,), jnp.float32)
                return 0
            lax.fori_loop(0, RPT // 16, zr, 0)
            for hh in range(NSUB):
                pltpu.sync_copy(stage.at[hh, pl.ds(rb, RPT)], rin)

                def ad(i, _):
                    sl = pl.ds(16 * i, 16)
                    racc[sl] = racc[sl] + rin[sl]
                    return 0
                lax.fori_loop(0, RPT // 16, ad, 0)
            pltpu.sync_copy(racc, out.at[pl.ds((t * 2 + cid) * NP + rb, RPT)])
            plsc.subcore_barrier()
            return 0
        lax.fori_loop(0, Tn, per_t, 0)

    return deg(srcf, wnegf)


def _prop_sc(tab, gidxf, sidxf, wnegf, Tn):
    NB = 3
    buf = lambda shp, dt: [pltpu.VMEM(shp, dt) for _ in range(NB)]

    @functools.partial(
        pl.kernel, mesh=_mesh(),
        out_type=jax.ShapeDtypeStruct((Tn * 2 * NP, F), jnp.float32),
        scratch_types=[
            pltpu.VMEM_SHARED((NP, F), jnp.float32),
            buf((C,), jnp.int32),
            buf((C,), jnp.int32),
            buf((C,), jnp.float32),
            buf((C, F), jnp.float32),
            [pltpu.SemaphoreType.DMA for _ in range(NB)],
            [pltpu.SemaphoreType.DMA for _ in range(NB)],
        ],
    )
    def prop(tabr, gidx, sidx, wneg, out, acc, gidxs, dstvs, wvs, rowss,
             semg, sems):
        cid = lax.axis_index("c")
        sid = lax.axis_index("s")
        wid = sid * NCORE + cid
        ebase0 = wid * (NCK * C)

        def fire_gather(t, k, b):
            eb = t * EP + ebase0 + k * C
            pltpu.sync_copy(gidx.at[pl.ds(eb, C)], gidxs[b])
            pltpu.sync_copy(sidx.at[pl.ds(eb, C)], dstvs[b])
            pltpu.sync_copy(wneg.at[pl.ds(eb, C)], wvs[b])
            goff = t * N
            for j in range(C // 16):
                sl = pl.ds(16 * j, 16)
                gidxs[b][sl] = gidxs[b][sl] + goff
            pltpu.async_copy(tabr.at[gidxs[b]], rowss[b], semg[b])

        def step_t(t, _):
            _zero_ref(rowss[0], C, F)
            for q in range(RPT // C):
                pltpu.sync_copy(rowss[0], acc.at[pl.ds(sid * RPT + q * C, C)])
            rem = RPT - (RPT // C) * C
            if rem:
                pltpu.sync_copy(
                    rowss[0].at[pl.ds(0, rem)],
                    acc.at[pl.ds(sid * RPT + (RPT // C) * C, rem)])
            plsc.subcore_barrier()
            for b in range(2):
                fire_gather(t, b, b)

            def outer(g, _):
                for b in range(NB):
                    k = 3 * g + b
                    bp = (b + 2) % 3
                    pltpu.make_async_copy(
                        tabr.at[gidxs[b]], rowss[b], semg[b]).wait()

                    def scale(gg, _):
                        wvec = wvs[b][pl.ds(16 * gg, 16)]
                        for l in range(16):
                            w = wvec[l]
                            e = 16 * gg + l
                            for j in range(F // 16):
                                sl = pl.ds(16 * j, 16)
                                rowss[b][e, sl] = rowss[b][e, sl] * w
                        return 0
                    lax.fori_loop(0, C // 16, scale, 0, unroll=2)

                    @pl.when(k + 2 < NCK)
                    def _():
                        @pl.when(k >= 1)
                        def _():
                            pltpu.make_async_copy(
                                rowss[bp], acc.at[dstvs[bp]],
                                sems[bp]).wait()
                        fire_gather(t, k + 2, bp)
                    pltpu.async_copy(rowss[b], acc.at[dstvs[b]], sems[b],
                                     add=True)
                return 0
            lax.fori_loop(0, NCK // 3, outer, 0)
            for b in range(NB):
                pltpu.make_async_copy(rowss[b], acc.at[dstvs[b]],
                                      sems[b]).wait()
            plsc.subcore_barrier()
            rb = sid * RPT
            pltpu.sync_copy(acc.at[pl.ds(rb, RPT)],
                            out.at[pl.ds((t * 2 + cid) * NP + rb, RPT)])
            plsc.subcore_barrier()
            return 0
        lax.fori_loop(0, Tn, step_t, 0)

    return prop(tab, gidxf, sidxf, wnegf)


def _wneg_tc(src2, dst2, ew2):
    def body(s_ref, d_ref, ew_ref, o_ref):
        o_ref[...] = jnp.where(s_ref[...] == d_ref[...], 0.0, -ew_ref[...])

    spec = pl.BlockSpec((8, WC), lambda i: (i, 0))
    return pl.pallas_call(
        body,
        grid=(WR // 8,),
        in_specs=[spec, spec, spec],
        out_specs=spec,
        out_shape=jax.ShapeDtypeStruct((WR, WC), jnp.float32),
    )(src2, dst2, ew2)


def _tabx_tc(x, dis3):
    def body(x_ref, d_ref, o_ref):
        o_ref[0] = x_ref[0] * d_ref[0]

    return pl.pallas_call(
        body,
        grid=(T, N // RB),
        in_specs=[
            pl.BlockSpec((1, RB, F), lambda t, i: (t, i, 0)),
            pl.BlockSpec((1, RB, 1), lambda t, i: (t, i, 0)),
        ],
        out_specs=pl.BlockSpec((1, RB, F), lambda t, i: (t, i, 0)),
        out_shape=jax.ShapeDtypeStruct((T, N, F), jnp.float32),
    )(x, dis3)


def _gx_tc(x, A, dis3, Wx0c, Wx1c, bcat):
    def body(x_ref, a0_ref, a1_ref, d_ref, w0_ref, w1_ref, b_ref,
             gzr_ref, gh_ref):
        a = (a0_ref[0] + a1_ref[0]) * d_ref[0]
        g = (lax.dot_general(x_ref[0], w0_ref[...], (((1,), (0,)), ((), ())),
                             precision=HP, preferred_element_type=jnp.float32)
             + lax.dot_general(a, w1_ref[...], (((1,), (0,)), ((), ())),
                               precision=HP,
                               preferred_element_type=jnp.float32)
             + b_ref[...])
        gzr_ref[0] = g[:, :256]
        gh_ref[0] = g[:, 256:]

    return pl.pallas_call(
        body,
        grid=(T, N // RB),
        in_specs=[
            pl.BlockSpec((1, RB, F), lambda t, i: (t, i, 0)),
            pl.BlockSpec((1, RB, F), lambda t, i: (2 * t, i, 0)),
            pl.BlockSpec((1, RB, F), lambda t, i: (2 * t + 1, i, 0)),
            pl.BlockSpec((1, RB, 1), lambda t, i: (t, i, 0)),
            pl.BlockSpec((F, 384), lambda t, i: (0, 0)),
            pl.BlockSpec((F, 384), lambda t, i: (0, 0)),
            pl.BlockSpec((1, 384), lambda t, i: (0, 0)),
        ],
        out_specs=[
            pl.BlockSpec((1, RB, 256), lambda t, i: (t, i, 0)),
            pl.BlockSpec((1, RB, F), lambda t, i: (t, i, 0)),
        ],
        out_shape=[
            jax.ShapeDtypeStruct((T, N, 256), jnp.float32),
            jax.ShapeDtypeStruct((T, N, F), jnp.float32),
        ],
    )(x, A, A, dis3, Wx0c, Wx1c, bcat)


def _gates_tc(h, B, Gzr_t, dis3_t, Wzr0, Wzr1):
    def body(h_ref, b0_ref, b1_ref, g_ref, d_ref, w0_ref, w1_ref,
             z_ref, c_ref, ct_ref):
        hb = h_ref[...]
        d = d_ref[...]
        bb = (b0_ref[0] + b1_ref[0]) * d
        pre = (g_ref[...]
               + lax.dot_general(hb, w0_ref[...], (((1,), (0,)), ((), ())),
                                 precision=HP,
                                 preferred_element_type=jnp.float32)
               + lax.dot_general(bb, w1_ref[...], (((1,), (0,)), ((), ())),
                                 precision=HP,
                                 preferred_element_type=jnp.float32))
        zr = jax.nn.sigmoid(pre)
        z = zr[:, :F]
        r = zr[:, F:]
        c = hb * r
        z_ref[...] = z
        c_ref[...] = c
        ct_ref[...] = c * d

    return pl.pallas_call(
        body,
        grid=(N // RB,),
        in_specs=[
            pl.BlockSpec((RB, F), lambda i: (i, 0)),
            pl.BlockSpec((1, RB, F), lambda i: (0, i, 0)),
            pl.BlockSpec((1, RB, F), lambda i: (1, i, 0)),
            pl.BlockSpec((RB, 256), lambda i: (i, 0)),
            pl.BlockSpec((RB, 1), lambda i: (i, 0)),
            pl.BlockSpec((F, 256), lambda i: (0, 0)),
            pl.BlockSpec((F, 256), lambda i: (0, 0)),
        ],
        out_specs=[
            pl.BlockSpec((RB, F), lambda i: (i, 0)),
            pl.BlockSpec((RB, F), lambda i: (i, 0)),
            pl.BlockSpec((RB, F), lambda i: (i, 0)),
        ],
        out_shape=[
            jax.ShapeDtypeStruct((N, F), jnp.float32),
            jax.ShapeDtypeStruct((N, F), jnp.float32),
            jax.ShapeDtypeStruct((N, F), jnp.float32),
        ],
    )(h, B, B, Gzr_t, dis3_t, Wzr0, Wzr1)


def _upd_tc(h, z, c, D, Gh_t, dis3_t, dis3n, Whh0, Whh1):
    def body(h_ref, z_ref, c_ref, d0_ref, d1_ref, g_ref, dt_ref, dn_ref,
             w0_ref, w1_ref, hn_ref, ht_ref):
        db = (d0_ref[0] + d1_ref[0]) * dt_ref[...]
        pre = (g_ref[...]
               + lax.dot_general(c_ref[...], w0_ref[...],
                                 (((1,), (0,)), ((), ())), precision=HP,
                                 preferred_element_type=jnp.float32)
               + lax.dot_general(db, w1_ref[...], (((1,), (0,)), ((), ())),
                                 precision=HP,
                                 preferred_element_type=jnp.float32))
        htil = jnp.tanh(pre)
        zb = z_ref[...]
        hn = zb * h_ref[...] + (1.0 - zb) * htil
        hn_ref[...] = hn
        ht_ref[...] = hn * dn_ref[...]

    return pl.pallas_call(
        body,
        grid=(N // RB,),
        in_specs=[
            pl.BlockSpec((RB, F), lambda i: (i, 0)),
            pl.BlockSpec((RB, F), lambda i: (i, 0)),
            pl.BlockSpec((RB, F), lambda i: (i, 0)),
            pl.BlockSpec((1, RB, F), lambda i: (0, i, 0)),
            pl.BlockSpec((1, RB, F), lambda i: (1, i, 0)),
            pl.BlockSpec((RB, F), lambda i: (i, 0)),
            pl.BlockSpec((RB, 1), lambda i: (i, 0)),
            pl.BlockSpec((RB, 1), lambda i: (i, 0)),
            pl.BlockSpec((F, F), lambda i: (0, 0)),
            pl.BlockSpec((F, F), lambda i: (0, 0)),
        ],
        out_specs=[
            pl.BlockSpec((RB, F), lambda i: (i, 0)),
            pl.BlockSpec((RB, F), lambda i: (i, 0)),
        ],
        out_shape=[
            jax.ShapeDtypeStruct((N, F), jnp.float32),
            jax.ShapeDtypeStruct((N, F), jnp.float32),
        ],
    )(h, z, c, D, D, Gh_t, dis3_t, dis3n, Whh0, Whh1)


def _fin_tc(h, Wlin, blin2):
    def body(h_ref, w_ref, b_ref, o_ref):
        o_ref[...] = lax.dot_general(
            h_ref[...], w_ref[...], (((1,), (0,)), ((), ())), precision=HP,
            preferred_element_type=jnp.float32) + b_ref[0, 0]

    return pl.pallas_call(
        body,
        grid=(N // RB,),
        in_specs=[
            pl.BlockSpec((RB, F), lambda i: (i, 0)),
            pl.BlockSpec((F, 1), lambda i: (0, 0)),
            pl.BlockSpec((1, 1), lambda i: (0, 0)),
        ],
        out_specs=pl.BlockSpec((RB, 1), lambda i: (i, 0)),
        out_shape=jax.ShapeDtypeStruct((N, 1), jnp.float32),
    )(h, Wlin, blin2)


def kernel(x, edge_index, edge_weight, Wxz, bxz, Whz, bhz, Wxr, bxr, Whr, bhr,
           Wxh, bxh, Whh, bhh, Wlin, blin):
    src = edge_index[:, 0, :]
    dst = edge_index[:, 1, :]
    wneg = _wneg_tc(src.reshape(WR, WC), dst.reshape(WR, WC),
                    edge_weight.reshape(WR, WC)).reshape(T, E)
    pad = ((0, 0), (0, EP - E))
    srcf = jnp.pad(src, pad).reshape(-1)
    dstf = jnp.pad(dst, pad).reshape(-1)
    wnegf = jnp.pad(wneg, pad).reshape(-1)

    ones_tab = jnp.ones((T * N, F), jnp.float32)
    degp = _prop_sc(ones_tab, jnp.zeros_like(srcf), srcf, wnegf, T)
    degp = degp.reshape(T, 2, NP, F)[..., 0]
    deg = -(degp[:, 0] + degp[:, 1])
    dis = jnp.where(deg > 0, lax.rsqrt(jnp.where(deg > 0, deg, 1.0)), 0.0)
    dis3 = dis[:, :N, None]

    xtab = _tabx_tc(x, dis3).reshape(T * N, F)
    A = _prop_sc(xtab, srcf, dstf, wnegf, T).reshape(2 * T, NP, F)

    Wx0c = jnp.concatenate([Wxz[0], Wxr[0], Wxh[0]], axis=1)
    Wx1c = jnp.concatenate([Wxz[1], Wxr[1], Wxh[1]], axis=1)
    bcat = jnp.concatenate([bxz + bhz, bxr + bhr, bxh + bhh]).reshape(1, 384)
    Gzr, Gh = _gx_tc(x, A, dis3, Wx0c, Wx1c, bcat)

    Wzr0 = jnp.concatenate([Whz[0], Whr[0]], axis=1)
    Wzr1 = jnp.concatenate([Whz[1], Whr[1]], axis=1)

    h = jnp.zeros((N, F), jnp.float32)
    htab = jnp.zeros((N, F), jnp.float32)
    for t in range(T):
        s_t = srcf[t * EP:(t + 1) * EP]
        d_t = dstf[t * EP:(t + 1) * EP]
        w_t = wnegf[t * EP:(t + 1) * EP]
        B = _prop_sc(htab, s_t, d_t, w_t, 1).reshape(2, NP, F)
        z, c, ctab = _gates_tc(h, B, Gzr[t], dis3[t], Wzr0, Wzr1)
        D = _prop_sc(ctab, s_t, d_t, w_t, 1).reshape(2, NP, F)
        h, htab = _upd_tc(h, z, c, D, Gh[t], dis3[t], dis3[(t + 1) % T],
                          Whh[0], Whh[1])

    y2 = _fin_tc(h, Wlin, blin.reshape(1, 1))
    return (y2[:, 0], h)


# speedup vs baseline: 9.9040x; 9.9040x over previous
import functools

import jax
import jax.numpy as jnp
from jax import lax
from jax.experimental import pallas as pl
from jax.experimental.pallas import tpu as pltpu
from jax.experimental.pallas import tpu_sc as plsc

T = 12
N = 10000
E = 320000
F = 128
NP = 10240
NCORE = 2
NSUB = 16
NW = NCORE * NSUB
C = 96
NCK = 105
EP = NW * C * NCK
EDT = EP // NW
RPT = NP // NSUB
RB = 400
WC = 2000
WR = T * E // WC
HP = lax.Precision.HIGHEST


def _mesh():
    return plsc.VectorSubcoreMesh(core_axis_name="c", subcore_axis_name="s")


def _zero_ref(ref, nrow, ncol):
    def z(i, _):
        for j in range(ncol // 16):
            ref[i, pl.ds(16 * j, 16)] = jnp.zeros((16,), jnp.float32)
        return 0
    lax.fori_loop(0, nrow, z, 0)


def _deg_sc(srcf, wnegf, Tn):
    @functools.partial(
        pl.kernel, mesh=_mesh(),
        out_type=jax.ShapeDtypeStruct((Tn * 2 * NP,), jnp.float32),
        scratch_types=[
            pltpu.VMEM_SHARED((NSUB, NP), jnp.float32),
            pltpu.VMEM((NP + 16,), jnp.float32),
            pltpu.VMEM((EDT,), jnp.int32),
            pltpu.VMEM((EDT,), jnp.float32),
            pltpu.VMEM((RPT,), jnp.float32),
            pltpu.VMEM((RPT,), jnp.float32),
        ],
    )
    def deg(src, wneg, out, stage, hist, idxv, wv, racc, rin):
        cid = lax.axis_index("c")
        sid = lax.axis_index("s")
        wid = sid * NCORE + cid
        m0 = (lax.iota(jnp.int32, 16) == 0).astype(jnp.float32)

        def per_t(t, _):
            def zh(i, _):
                hist[pl.ds(16 * i, 16)] = jnp.zeros((16,), jnp.float32)
                return 0
            lax.fori_loop(0, (NP + 16) // 16, zh, 0)
            base = t * EP + wid * EDT
            pltpu.sync_copy(src.at[pl.ds(base, EDT)], idxv)
            pltpu.sync_copy(wneg.at[pl.ds(base, EDT)], wv)

            def accg(g, _):
                ivec = idxv[pl.ds(16 * g, 16)]
                wvec = wv[pl.ds(16 * g, 16)]
                for l in range(16):
                    ix = ivec[l]
                    hist[pl.ds(ix, 16)] = hist[pl.ds(ix, 16)] + wvec[l] * m0
                return 0
            lax.fori_loop(0, EDT // 16, accg, 0)
            pltpu.sync_copy(hist.at[pl.ds(0, NP)], stage.at[sid])
            plsc.subcore_barrier()
            rb = sid * RPT

            def zr(i, _):
                racc[pl.ds(16 * i, 16)] = jnp.zeros((16,), jnp.float32)
                return 0
            lax.fori_loop(0, RPT // 16, zr, 0)
            for hh in range(NSUB):
                pltpu.sync_copy(stage.at[hh, pl.ds(rb, RPT)], rin)

                def ad(i, _):
                    sl = pl.ds(16 * i, 16)
                    racc[sl] = racc[sl] + rin[sl]
                    return 0
                lax.fori_loop(0, RPT // 16, ad, 0)
            pltpu.sync_copy(racc, out.at[pl.ds((t * 2 + cid) * NP + rb, RPT)])
            plsc.subcore_barrier()
            return 0
        lax.fori_loop(0, Tn, per_t, 0)

    return deg(srcf, wnegf)


def _prop_sc(tab, gidxf, sidxf, wnegf, Tn):
    NB = 3
    buf = lambda shp, dt: [pltpu.VMEM(shp, dt) for _ in range(NB)]

    @functools.partial(
        pl.kernel, mesh=_mesh(),
        out_type=jax.ShapeDtypeStruct((Tn * 2 * NP, F), jnp.float32),
        scratch_types=[
            pltpu.VMEM_SHARED((NP, F), jnp.float32),
            buf((C,), jnp.int32),
            buf((C,), jnp.int32),
            buf((C,), jnp.float32),
            buf((C, F), jnp.float32),
            [pltpu.SemaphoreType.DMA for _ in range(NB)],
            [pltpu.SemaphoreType.DMA for _ in range(NB)],
        ],
    )
    def prop(tabr, gidx, sidx, wneg, out, acc, gidxs, dstvs, wvs, rowss,
             semg, sems):
        cid = lax.axis_index("c")
        sid = lax.axis_index("s")
        wid = sid * NCORE + cid
        ebase0 = wid * (NCK * C)

        def fire_gather(t, k, b):
            eb = t * EP + ebase0 + k * C
            pltpu.sync_copy(gidx.at[pl.ds(eb, C)], gidxs[b])
            pltpu.sync_copy(sidx.at[pl.ds(eb, C)], dstvs[b])
            pltpu.sync_copy(wneg.at[pl.ds(eb, C)], wvs[b])
            goff = t * N
            for j in range(C // 16):
                sl = pl.ds(16 * j, 16)
                gidxs[b][sl] = gidxs[b][sl] + goff
            pltpu.async_copy(tabr.at[gidxs[b]], rowss[b], semg[b])

        def step_t(t, _):
            _zero_ref(rowss[0], C, F)
            for q in range(RPT // C):
                pltpu.sync_copy(rowss[0], acc.at[pl.ds(sid * RPT + q * C, C)])
            rem = RPT - (RPT // C) * C
            if rem:
                pltpu.sync_copy(
                    rowss[0].at[pl.ds(0, rem)],
                    acc.at[pl.ds(sid * RPT + (RPT // C) * C, rem)])
            plsc.subcore_barrier()
            for b in range(2):
                fire_gather(t, b, b)

            def outer(g, _):
                for b in range(NB):
                    k = 3 * g + b
                    bp = (b + 2) % 3
                    pltpu.make_async_copy(
                        tabr.at[gidxs[b]], rowss[b], semg[b]).wait()

                    def scale(gg, _):
                        wvec = wvs[b][pl.ds(16 * gg, 16)]
                        for l in range(16):
                            w = wvec[l]
                            e = 16 * gg + l
                            for j in range(F // 16):
                                sl = pl.ds(16 * j, 16)
                                rowss[b][e, sl] = rowss[b][e, sl] * w
                        return 0
                    lax.fori_loop(0, C // 16, scale, 0, unroll=2)

                    @pl.when(k + 2 < NCK)
                    def _():
                        @pl.when(k >= 1)
                        def _():
                            pltpu.make_async_copy(
                                rowss[bp], acc.at[dstvs[bp]],
                                sems[bp]).wait()
                        fire_gather(t, k + 2, bp)
                    pltpu.async_copy(rowss[b], acc.at[dstvs[b]], sems[b],
                                     add=True)
                return 0
            lax.fori_loop(0, NCK // 3, outer, 0)
            for b in range(NB):
                pltpu.make_async_copy(rowss[b], acc.at[dstvs[b]],
                                      sems[b]).wait()
            plsc.subcore_barrier()
            rb = sid * RPT
            pltpu.sync_copy(acc.at[pl.ds(rb, RPT)],
                            out.at[pl.ds((t * 2 + cid) * NP + rb, RPT)])
            plsc.subcore_barrier()
            return 0
        lax.fori_loop(0, Tn, step_t, 0)

    return prop(tab, gidxf, sidxf, wnegf)


def _wneg_tc(src2, dst2, ew2):
    def body(s_ref, d_ref, ew_ref, o_ref):
        o_ref[...] = jnp.where(s_ref[...] == d_ref[...], 0.0, -ew_ref[...])

    spec = pl.BlockSpec((8, WC), lambda i: (i, 0))
    return pl.pallas_call(
        body,
        grid=(WR // 8,),
        in_specs=[spec, spec, spec],
        out_specs=spec,
        out_shape=jax.ShapeDtypeStruct((WR, WC), jnp.float32),
    )(src2, dst2, ew2)


def _tabx_tc(x, dis3):
    def body(x_ref, d_ref, o_ref):
        o_ref[0] = x_ref[0] * d_ref[0]

    return pl.pallas_call(
        body,
        grid=(T, N // RB),
        in_specs=[
            pl.BlockSpec((1, RB, F), lambda t, i: (t, i, 0)),
            pl.BlockSpec((1, RB, 1), lambda t, i: (t, i, 0)),
        ],
        out_specs=pl.BlockSpec((1, RB, F), lambda t, i: (t, i, 0)),
        out_shape=jax.ShapeDtypeStruct((T, N, F), jnp.float32),
    )(x, dis3)


def _gx_tc(x, A, dis3, Wx0c, Wx1c, bcat):
    def body(x_ref, a0_ref, a1_ref, d_ref, w0_ref, w1_ref, b_ref,
             gzr_ref, gh_ref):
        a = (a0_ref[0] + a1_ref[0]) * d_ref[0]
        g = (lax.dot_general(x_ref[0], w0_ref[...], (((1,), (0,)), ((), ())),
                             precision=HP, preferred_element_type=jnp.float32)
             + lax.dot_general(a, w1_ref[...], (((1,), (0,)), ((), ())),
                               precision=HP,
                               preferred_element_type=jnp.float32)
             + b_ref[...])
        gzr_ref[0] = g[:, :256]
        gh_ref[0] = g[:, 256:]

    return pl.pallas_call(
        body,
        grid=(T, N // RB),
        in_specs=[
            pl.BlockSpec((1, RB, F), lambda t, i: (t, i, 0)),
            pl.BlockSpec((1, RB, F), lambda t, i: (2 * t, i, 0)),
            pl.BlockSpec((1, RB, F), lambda t, i: (2 * t + 1, i, 0)),
            pl.BlockSpec((1, RB, 1), lambda t, i: (t, i, 0)),
            pl.BlockSpec((F, 384), lambda t, i: (0, 0)),
            pl.BlockSpec((F, 384), lambda t, i: (0, 0)),
            pl.BlockSpec((1, 384), lambda t, i: (0, 0)),
        ],
        out_specs=[
            pl.BlockSpec((1, RB, 256), lambda t, i: (t, i, 0)),
            pl.BlockSpec((1, RB, F), lambda t, i: (t, i, 0)),
        ],
        out_shape=[
            jax.ShapeDtypeStruct((T, N, 256), jnp.float32),
            jax.ShapeDtypeStruct((T, N, F), jnp.float32),
        ],
    )(x, A, A, dis3, Wx0c, Wx1c, bcat)


def _gates_tc(h, B, Gzr_t, dis3_t, Wzr0, Wzr1):
    def body(h_ref, b0_ref, b1_ref, g_ref, d_ref, w0_ref, w1_ref,
             z_ref, c_ref, ct_ref):
        hb = h_ref[...]
        d = d_ref[...]
        bb = (b0_ref[0] + b1_ref[0]) * d
        pre = (g_ref[...]
               + lax.dot_general(hb, w0_ref[...], (((1,), (0,)), ((), ())),
                                 precision=HP,
                                 preferred_element_type=jnp.float32)
               + lax.dot_general(bb, w1_ref[...], (((1,), (0,)), ((), ())),
                                 precision=HP,
                                 preferred_element_type=jnp.float32))
        zr = jax.nn.sigmoid(pre)
        z = zr[:, :F]
        r = zr[:, F:]
        c = hb * r
        z_ref[...] = z
        c_ref[...] = c
        ct_ref[...] = c * d

    return pl.pallas_call(
        body,
        grid=(N // RB,),
        in_specs=[
            pl.BlockSpec((RB, F), lambda i: (i, 0)),
            pl.BlockSpec((1, RB, F), lambda i: (0, i, 0)),
            pl.BlockSpec((1, RB, F), lambda i: (1, i, 0)),
            pl.BlockSpec((RB, 256), lambda i: (i, 0)),
            pl.BlockSpec((RB, 1), lambda i: (i, 0)),
            pl.BlockSpec((F, 256), lambda i: (0, 0)),
            pl.BlockSpec((F, 256), lambda i: (0, 0)),
        ],
        out_specs=[
            pl.BlockSpec((RB, F), lambda i: (i, 0)),
            pl.BlockSpec((RB, F), lambda i: (i, 0)),
            pl.BlockSpec((RB, F), lambda i: (i, 0)),
        ],
        out_shape=[
            jax.ShapeDtypeStruct((N, F), jnp.float32),
            jax.ShapeDtypeStruct((N, F), jnp.float32),
            jax.ShapeDtypeStruct((N, F), jnp.float32),
        ],
    )(h, B, B, Gzr_t, dis3_t, Wzr0, Wzr1)


def _upd_tc(h, z, c, D, Gh_t, dis3_t, dis3n, Whh0, Whh1):
    def body(h_ref, z_ref, c_ref, d0_ref, d1_ref, g_ref, dt_ref, dn_ref,
             w0_ref, w1_ref, hn_ref, ht_ref):
        db = (d0_ref[0] + d1_ref[0]) * dt_ref[...]
        pre = (g_ref[...]
               + lax.dot_general(c_ref[...], w0_ref[...],
                                 (((1,), (0,)), ((), ())), precision=HP,
                                 preferred_element_type=jnp.float32)
               + lax.dot_general(db, w1_ref[...], (((1,), (0,)), ((), ())),
                                 precision=HP,
                                 preferred_element_type=jnp.float32))
        htil = jnp.tanh(pre)
        zb = z_ref[...]
        hn = zb * h_ref[...] + (1.0 - zb) * htil
        hn_ref[...] = hn
        ht_ref[...] = hn * dn_ref[...]

    return pl.pallas_call(
        body,
        grid=(N // RB,),
        in_specs=[
            pl.BlockSpec((RB, F), lambda i: (i, 0)),
            pl.BlockSpec((RB, F), lambda i: (i, 0)),
            pl.BlockSpec((RB, F), lambda i: (i, 0)),
            pl.BlockSpec((1, RB, F), lambda i: (0, i, 0)),
            pl.BlockSpec((1, RB, F), lambda i: (1, i, 0)),
            pl.BlockSpec((RB, F), lambda i: (i, 0)),
            pl.BlockSpec((RB, 1), lambda i: (i, 0)),
            pl.BlockSpec((RB, 1), lambda i: (i, 0)),
            pl.BlockSpec((F, F), lambda i: (0, 0)),
            pl.BlockSpec((F, F), lambda i: (0, 0)),
        ],
        out_specs=[
            pl.BlockSpec((RB, F), lambda i: (i, 0)),
            pl.BlockSpec((RB, F), lambda i: (i, 0)),
        ],
        out_shape=[
            jax.ShapeDtypeStruct((N, F), jnp.float32),
            jax.ShapeDtypeStruct((N, F), jnp.float32),
        ],
    )(h, z, c, D, D, Gh_t, dis3_t, dis3n, Whh0, Whh1)


def _fin_tc(h, Wlin, blin2):
    def body(h_ref, w_ref, b_ref, o_ref):
        o_ref[...] = lax.dot_general(
            h_ref[...], w_ref[...], (((1,), (0,)), ((), ())), precision=HP,
            preferred_element_type=jnp.float32) + b_ref[0, 0]

    return pl.pallas_call(
        body,
        grid=(N // RB,),
        in_specs=[
            pl.BlockSpec((RB, F), lambda i: (i, 0)),
            pl.BlockSpec((F, 1), lambda i: (0, 0)),
            pl.BlockSpec((1, 1), lambda i: (0, 0)),
        ],
        out_specs=pl.BlockSpec((RB, 1), lambda i: (i, 0)),
        out_shape=jax.ShapeDtypeStruct((N, 1), jnp.float32),
    )(h, Wlin, blin2)


def kernel(x, edge_index, edge_weight, Wxz, bxz, Whz, bhz, Wxr, bxr, Whr, bhr,
           Wxh, bxh, Whh, bhh, Wlin, blin):
    src = edge_index[:, 0, :]
    dst = edge_index[:, 1, :]
    wneg = _wneg_tc(src.reshape(WR, WC), dst.reshape(WR, WC),
                    edge_weight.reshape(WR, WC)).reshape(T, E)
    pad = ((0, 0), (0, EP - E))
    srcf = jnp.pad(src, pad).reshape(-1)
    dstf = jnp.pad(dst, pad).reshape(-1)
    wnegf = jnp.pad(wneg, pad).reshape(-1)

    ones_tab = jnp.ones((T * N, F), jnp.float32)
    degp = _prop_sc(ones_tab, srcf, srcf, wnegf, T)
    degp = degp.reshape(T, 2, NP, F)[..., 0]
    deg = -(degp[:, 0] + degp[:, 1])
    dis = jnp.where(deg > 0, lax.rsqrt(jnp.where(deg > 0, deg, 1.0)), 0.0)
    dis3 = dis[:, :N, None]

    xtab = _tabx_tc(x, dis3).reshape(T * N, F)
    A = _prop_sc(xtab, srcf, dstf, wnegf, T).reshape(2 * T, NP, F)

    Wx0c = jnp.concatenate([Wxz[0], Wxr[0], Wxh[0]], axis=1)
    Wx1c = jnp.concatenate([Wxz[1], Wxr[1], Wxh[1]], axis=1)
    bcat = jnp.concatenate([bxz + bhz, bxr + bhr, bxh + bhh]).reshape(1, 384)
    Gzr, Gh = _gx_tc(x, A, dis3, Wx0c, Wx1c, bcat)

    Wzr0 = jnp.concatenate([Whz[0], Whr[0]], axis=1)
    Wzr1 = jnp.concatenate([Whz[1], Whr[1]], axis=1)

    h = jnp.zeros((N, F), jnp.float32)
    htab = jnp.zeros((N, F), jnp.float32)
    for t in range(T):
        s_t = srcf[t * EP:(t + 1) * EP]
        d_t = dstf[t * EP:(t + 1) * EP]
        w_t = wnegf[t * EP:(t + 1) * EP]
        B = _prop_sc(htab, s_t, d_t, w_t, 1).reshape(2, NP, F)
        z, c, ctab = _gates_tc(h, B, Gzr[t], dis3[t], Wzr0, Wzr1)
        D = _prop_sc(ctab, s_t, d_t, w_t, 1).reshape(2, NP, F)
        h, htab = _upd_tc(h, z, c, D, Gh[t], dis3[t], dis3[(t + 1) % T],
                          Whh[0], Whh[1])

    y2 = _fin_tc(h, Wlin, blin.reshape(1, 1))
    return (y2[:, 0], h)
